# baseline (speedup 1.0000x reference)
import functools
import math

import jax
import jax.numpy as jnp
from jax import lax
from jax.experimental import pallas as pl
from jax.experimental.pallas import tpu as pltpu
from jax.experimental.pallas import tpu_sc as plsc

B = 16384
T = 50
D_MODEL = 64
LANES = 16
NUM_CORES = 2
NUM_SUBCORES = 16
NUM_WORKERS = NUM_CORES * NUM_SUBCORES
BW = 128
B_PER_W = B // NUM_WORKERS
BLKS = B_PER_W // BW
NBUF = 3
SCALE = math.sqrt(D_MODEL)


def _sc_embedding(x, lut2):
    n_chunks = BLKS * T

    mesh = plsc.VectorSubcoreMesh(
        core_axis_name="c", subcore_axis_name="s",
        num_cores=NUM_CORES, num_subcores=NUM_SUBCORES,
    )

    @functools.partial(
        pl.kernel,
        out_type=jax.ShapeDtypeStruct((T, D_MODEL, B), jnp.float32),
        mesh=mesh,
        scratch_types=[
            pltpu.VMEM((B_PER_W * T,), jnp.int32),
            pltpu.VMEM((NBUF * BW,), jnp.int32),
            pltpu.VMEM((NBUF * BW, D_MODEL), jnp.float32),
            pltpu.VMEM((NBUF * D_MODEL, BW), jnp.float32),
            [pltpu.SemaphoreType.DMA] * NBUF,
            [pltpu.SemaphoreType.DMA] * NBUF,
        ],
        compiler_params=pltpu.CompilerParams(
            needs_layout_passes=False, use_tc_tiling_on_sc=False),
    )
    def body(x_hbm, lut_hbm, out_hbm, xblk_v, ridx_v, rows_v, trans_v,
             sem_g, sem_o):
        wid = lax.axis_index("s") * NUM_CORES + lax.axis_index("c")
        g0 = pl.multiple_of(wid * B_PER_W, B_PER_W)

        iota16 = lax.iota(jnp.int32, LANES)
        iota_t = iota16 * T

        def stage_and_fire(cur, b):
            blk = cur // T
            t = cur - blk * T
            for g in range(BW // LANES):
                flat = iota_t + ((blk * BW + g * LANES) * T + t)
                v = plsc.load_gather(xblk_v, [flat])
                ridx_v[pl.ds(b * BW + g * LANES, LANES)] = v
            pltpu.async_copy(
                lut_hbm.at[ridx_v.at[pl.ds(b * BW, BW)]],
                rows_v.at[pl.ds(b * BW, BW)],
                sem_g[b],
            )

        def drain_gather(b):
            pltpu.make_async_copy(
                lut_hbm.at[pl.ds(0, BW)],
                rows_v.at[pl.ds(b * BW, BW)],
                sem_g[b],
            ).wait()

        def drain_out(b):
            pltpu.make_async_copy(
                out_hbm.at[0, pl.ds(0, D_MODEL), pl.ds(0, BW)],
                trans_v.at[pl.ds(b * D_MODEL, D_MODEL)],
                sem_o[b],
            ).wait()

        def transpose_scale(b):
            ngrp = BW // LANES
            rowvecs = [iota16 + (b * BW + g * LANES) for g in range(ngrp)]
            zero16 = iota16 * 0

            @plsc.parallel_loop(0, D_MODEL, unroll=4)
            def _d(d):
                dvec = zero16 + d
                for g in range(ngrp):
                    v = plsc.load_gather(rows_v, [rowvecs[g], dvec])
                    trans_v[b * D_MODEL + d, pl.ds(g * LANES, LANES)] = v * SCALE

        def out_write(cur, b):
            blk = cur // T
            t = cur - blk * T
            b0 = pl.multiple_of(g0 + blk * BW, BW)
            pltpu.async_copy(
                trans_v.at[pl.ds(b * D_MODEL, D_MODEL)],
                out_hbm.at[t, pl.ds(0, D_MODEL), pl.ds(b0, BW)],
                sem_o[b],
            )

        f0 = pl.multiple_of(g0 * T, B_PER_W * T)
        pltpu.sync_copy(x_hbm.at[pl.ds(f0, B_PER_W * T)], xblk_v)

        stage_and_fire(0, 0)
        stage_and_fire(1, 1)

        @pl.loop(0, n_chunks, step=NBUF)
        def _outer(s):
            for b in range(NBUF):
                cur = s + b
                bf = (b + 2) % NBUF

                @pl.when(jnp.logical_and(cur + 2 < n_chunks, cur >= 1))
                def _():
                    drain_out(bf)
                    stage_and_fire(cur + 2, bf)

                @pl.when(jnp.logical_and(cur + 2 < n_chunks, cur < 1))
                def _():
                    stage_and_fire(cur + 2, bf)

                @pl.when(cur < n_chunks)
                def _():
                    drain_gather(b)
                    transpose_scale(b)
                    out_write(cur, b)

        for b in range(NBUF):
            drain_out(b)

    return body(x, lut2)


def kernel(x, lut):
    out_t = _sc_embedding(x.reshape(-1).astype(jnp.int32), lut)
    return out_t.transpose(2, 0, 1)


# speedup vs baseline: 1.1561x; 1.1561x over previous
import functools
import math

import jax
import jax.numpy as jnp
from jax import lax
from jax.experimental import pallas as pl
from jax.experimental.pallas import tpu as pltpu
from jax.experimental.pallas import tpu_sc as plsc

B = 16384
T = 50
D_MODEL = 64
LANES = 16
NUM_CORES = 2
NUM_SUBCORES = 16
NUM_WORKERS = NUM_CORES * NUM_SUBCORES
BW = 128
B_PER_W = B // NUM_WORKERS
BLKS = B_PER_W // BW
NBUF = 4
AHEAD = NBUF - 1
SCALE = math.sqrt(D_MODEL)


def _sc_embedding(x, lut2):
    n_chunks = BLKS * T

    mesh = plsc.VectorSubcoreMesh(
        core_axis_name="c", subcore_axis_name="s",
        num_cores=NUM_CORES, num_subcores=NUM_SUBCORES,
    )

    @functools.partial(
        pl.kernel,
        out_type=jax.ShapeDtypeStruct((T, D_MODEL, B), jnp.float32),
        mesh=mesh,
        scratch_types=[
            pltpu.VMEM((B_PER_W * T,), jnp.int32),
            pltpu.VMEM((NBUF * BW,), jnp.int32),
            pltpu.VMEM((NBUF * BW,), jnp.int32),
            pltpu.VMEM((NBUF * BW, 2 * D_MODEL), jnp.float32),
            pltpu.VMEM((NBUF * D_MODEL, BW), jnp.float32),
            [pltpu.SemaphoreType.DMA] * NBUF,
            [pltpu.SemaphoreType.DMA] * NBUF,
        ],
        compiler_params=pltpu.CompilerParams(needs_layout_passes=False),
    )
    def body(x_hbm, lut_hbm, out_hbm, xblk_v, ridx_v, par_v, rows_v, trans_v,
             sem_g, sem_o):
        wid = lax.axis_index("s") * NUM_CORES + lax.axis_index("c")
        g0 = pl.multiple_of(wid * B_PER_W, B_PER_W)

        iota16 = lax.iota(jnp.int32, LANES)
        iota_t = iota16 * T

        def stage_and_fire(cur, b):
            blk = cur // T
            t = cur - blk * T
            for g in range(BW // LANES):
                flat = iota_t + ((blk * BW + g * LANES) * T + t)
                v = plsc.load_gather(xblk_v, [flat])
                sl = pl.ds(b * BW + g * LANES, LANES)
                ridx_v[sl] = lax.shift_right_logical(v, 1)
                par_v[sl] = (v & 1) * D_MODEL
            pltpu.async_copy(
                lut_hbm.at[ridx_v.at[pl.ds(b * BW, BW)]],
                rows_v.at[pl.ds(b * BW, BW)],
                sem_g[b],
            )

        def drain_gather(b):
            pltpu.make_async_copy(
                lut_hbm.at[pl.ds(0, BW)],
                rows_v.at[pl.ds(b * BW, BW)],
                sem_g[b],
            ).wait()

        def drain_out(b):
            pltpu.make_async_copy(
                out_hbm.at[0, pl.ds(0, D_MODEL), pl.ds(0, BW)],
                trans_v.at[pl.ds(b * D_MODEL, D_MODEL)],
                sem_o[b],
            ).wait()

        def transpose_scale(b):
            ngrp = BW // LANES
            rowvecs = [iota16 + (b * BW + g * LANES) for g in range(ngrp)]
            parvecs = [par_v[pl.ds(b * BW + g * LANES, LANES)] for g in range(ngrp)]

            @plsc.parallel_loop(0, D_MODEL, unroll=4)
            def _d(d):
                for g in range(ngrp):
                    v = plsc.load_gather(rows_v, [rowvecs[g], parvecs[g] + d])
                    trans_v[b * D_MODEL + d, pl.ds(g * LANES, LANES)] = v * SCALE

        def out_write(cur, b):
            blk = cur // T
            t = cur - blk * T
            b0 = pl.multiple_of(g0 + blk * BW, BW)
            pltpu.async_copy(
                trans_v.at[pl.ds(b * D_MODEL, D_MODEL)],
                out_hbm.at[t, pl.ds(0, D_MODEL), pl.ds(b0, BW)],
                sem_o[b],
            )

        f0 = pl.multiple_of(g0 * T, B_PER_W * T)
        pltpu.sync_copy(x_hbm.at[pl.ds(f0, B_PER_W * T)], xblk_v)

        for p0 in range(AHEAD):
            stage_and_fire(p0, p0)

        @pl.loop(0, n_chunks + NBUF - 1 - ((n_chunks - 1) % NBUF), step=NBUF)
        def _outer(s):
            for b in range(NBUF):
                cur = s + b
                bf = (b + AHEAD) % NBUF

                @pl.when(jnp.logical_and(cur + AHEAD < n_chunks, cur >= 1))
                def _():
                    drain_out(bf)
                    stage_and_fire(cur + AHEAD, bf)

                @pl.when(jnp.logical_and(cur + AHEAD < n_chunks, cur < 1))
                def _():
                    stage_and_fire(cur + AHEAD, bf)

                @pl.when(cur < n_chunks)
                def _():
                    drain_gather(b)
                    transpose_scale(b)
                    out_write(cur, b)

        for b in range(NBUF):
            drain_out(b)

    return body(x, lut2)


def kernel(x, lut):
    lut2 = lut.reshape(-1, 2 * D_MODEL)
    out_t = _sc_embedding(x.reshape(-1).astype(jnp.int32), lut2)
    return out_t.transpose(2, 0, 1)


# speedup vs baseline: 1.1585x; 1.0021x over previous
import functools
import math

import jax
import jax.numpy as jnp
from jax import lax
from jax.experimental import pallas as pl
from jax.experimental.pallas import tpu as pltpu
from jax.experimental.pallas import tpu_sc as plsc

B = 16384
T = 50
D_MODEL = 64
LANES = 16
NUM_CORES = 2
NUM_SUBCORES = 16
NUM_WORKERS = NUM_CORES * NUM_SUBCORES
BW = 128
TPC = 2
CIDX = BW * TPC
B_PER_W = B // NUM_WORKERS
BLKS = B_PER_W // BW
NBUF = 2
AHEAD = NBUF - 1
SCALE = math.sqrt(D_MODEL)


def _sc_embedding(x, lut2):
    n_chunks = BLKS * (T // TPC)

    mesh = plsc.VectorSubcoreMesh(
        core_axis_name="c", subcore_axis_name="s",
        num_cores=NUM_CORES, num_subcores=NUM_SUBCORES,
    )

    @functools.partial(
        pl.kernel,
        out_type=jax.ShapeDtypeStruct((T, D_MODEL, B), jnp.float32),
        mesh=mesh,
        scratch_types=[
            pltpu.VMEM((B_PER_W * T,), jnp.int32),
            pltpu.VMEM((NBUF * CIDX,), jnp.int32),
            pltpu.VMEM((NBUF * CIDX,), jnp.int32),
            pltpu.VMEM((NBUF * CIDX, 2 * D_MODEL), jnp.float32),
            pltpu.VMEM((NBUF * TPC, D_MODEL, BW), jnp.float32),
            [pltpu.SemaphoreType.DMA] * NBUF,
            [pltpu.SemaphoreType.DMA] * NBUF,
        ],
        compiler_params=pltpu.CompilerParams(needs_layout_passes=False),
    )
    def body(x_hbm, lut_hbm, out_hbm, xblk_v, ridx_v, par_v, rows_v, trans_v,
             sem_g, sem_o):
        wid = lax.axis_index("s") * NUM_CORES + lax.axis_index("c")
        g0 = pl.multiple_of(wid * B_PER_W, B_PER_W)

        iota16 = lax.iota(jnp.int32, LANES)
        iota_t = iota16 * T

        def stage_and_fire(cur, b):
            blk = cur // (T // TPC)
            t0 = (cur - blk * (T // TPC)) * TPC
            for tt in range(TPC):
                for g in range(BW // LANES):
                    flat = iota_t + ((blk * BW + g * LANES) * T + t0 + tt)
                    v = plsc.load_gather(xblk_v, [flat])
                    sl = pl.ds(b * CIDX + tt * BW + g * LANES, LANES)
                    ridx_v[sl] = lax.shift_right_logical(v, 1)
                    par_v[sl] = (v & 1) * D_MODEL
            pltpu.async_copy(
                lut_hbm.at[ridx_v.at[pl.ds(b * CIDX, CIDX)]],
                rows_v.at[pl.ds(b * CIDX, CIDX)],
                sem_g[b],
            )

        def drain_gather(b):
            pltpu.make_async_copy(
                lut_hbm.at[pl.ds(0, CIDX)],
                rows_v.at[pl.ds(b * CIDX, CIDX)],
                sem_g[b],
            ).wait()

        def drain_out(b):
            pltpu.make_async_copy(
                out_hbm.at[pl.ds(0, TPC), pl.ds(0, D_MODEL), pl.ds(0, BW)],
                trans_v.at[pl.ds(b * TPC, TPC)],
                sem_o[b],
            ).wait()

        def transpose_scale(b):
            ngrp = BW // LANES
            for tt in range(TPC):
                base = b * CIDX + tt * BW
                rowvecs = [iota16 + (base + g * LANES) for g in range(ngrp)]
                parvecs = [par_v[pl.ds(base + g * LANES, LANES)] for g in range(ngrp)]

                @plsc.parallel_loop(0, D_MODEL, unroll=4)
                def _d(d):
                    for g in range(ngrp):
                        v = plsc.load_gather(rows_v, [rowvecs[g], parvecs[g] + d])
                        trans_v[b * TPC + tt, d, pl.ds(g * LANES, LANES)] = v * SCALE

        def out_write(cur, b):
            blk = cur // (T // TPC)
            t0 = (cur - blk * (T // TPC)) * TPC
            b0 = pl.multiple_of(g0 + blk * BW, BW)
            pltpu.async_copy(
                trans_v.at[pl.ds(b * TPC, TPC)],
                out_hbm.at[pl.ds(t0, TPC), pl.ds(0, D_MODEL), pl.ds(b0, BW)],
                sem_o[b],
            )

        f0 = pl.multiple_of(g0 * T, B_PER_W * T)
        pltpu.sync_copy(x_hbm.at[pl.ds(f0, B_PER_W * T)], xblk_v)

        for p0 in range(AHEAD):
            stage_and_fire(p0, p0)

        @pl.loop(0, n_chunks + NBUF - 1 - ((n_chunks - 1) % NBUF), step=NBUF)
        def _outer(s):
            for b in range(NBUF):
                cur = s + b
                bf = (b + AHEAD) % NBUF

                @pl.when(jnp.logical_and(cur + AHEAD < n_chunks, cur >= 1))
                def _():
                    drain_out(bf)
                    stage_and_fire(cur + AHEAD, bf)

                @pl.when(jnp.logical_and(cur + AHEAD < n_chunks, cur < 1))
                def _():
                    stage_and_fire(cur + AHEAD, bf)

                @pl.when(cur < n_chunks)
                def _():
                    drain_gather(b)
                    transpose_scale(b)
                    out_write(cur, b)

        for b in range(NBUF):
            drain_out(b)

    return body(x, lut2)


def kernel(x, lut):
    lut2 = lut.reshape(-1, 2 * D_MODEL)
    out_t = _sc_embedding(x.reshape(-1).astype(jnp.int32), lut2)
    return out_t.transpose(2, 0, 1)
